# baseline (speedup 1.0000x reference)
import functools

import jax
import jax.numpy as jnp
from jax.experimental import pallas as pl
from jax.experimental.pallas import tpu as pltpu

K = 8192
D = 256
N = 16384
BM = 256
NB = N // BM


def _vq_body(z_ref, cbt_ref, cb16_ref, zqst_ref, loss_ref, perp_ref,
             cbt16_s, c2_s, cnt_s, loss_s):
    i = pl.program_id(0)

    @pl.when(i == 0)
    def _init():
        cbt16_s[...] = cbt_ref[...].astype(jnp.bfloat16)
        c2_s[...] = jnp.sum(cbt_ref[...] * cbt_ref[...], axis=0, keepdims=True)
        cnt_s[...] = jnp.zeros_like(cnt_s)
        loss_s[...] = jnp.zeros_like(loss_s)

    zb = z_ref[...]
    zb16 = zb.astype(jnp.bfloat16)
    p2 = 2.0 * jnp.dot(zb16, cbt16_s[...], preferred_element_type=jnp.float32)
    a = jnp.sum(zb * zb, axis=1, keepdims=True)
    d = (a - p2) + c2_s[...]

    m = jnp.min(d, axis=1, keepdims=True)
    lanes = jax.lax.broadcasted_iota(jnp.int32, (BM, K), 1)
    idx = jnp.min(jnp.where(d == m, lanes, K), axis=1)

    oh = (lanes == idx[:, None]).astype(jnp.float32)
    cnt_s[...] += jnp.sum(oh, axis=0, keepdims=True)
    zq = jnp.dot(oh.astype(jnp.bfloat16), cb16_ref[...],
                 preferred_element_type=jnp.float32)

    zqst_ref[...] = zb + (zq - zb)
    loss_s[...] += jnp.sum((zq - zb) * (zq - zb))[None, None]

    @pl.when(i == NB - 1)
    def _fini():
        loss_ref[...] = 0.25 * loss_s[...] / (N * D)
        p = cnt_s[...] * (1.0 / N)
        perp_ref[...] = jnp.exp(-jnp.sum(p * jnp.log(p + 1e-05)))[None, None]


@functools.partial(jax.jit, static_argnames=())
def kernel(z, codebook):
    z2d = z.reshape(N, D)
    cbt = codebook.T
    cb16 = codebook.astype(jnp.bfloat16)

    zqst, loss, perp = pl.pallas_call(
        _vq_body,
        grid=(NB,),
        in_specs=[
            pl.BlockSpec((BM, D), lambda i: (i, 0)),
            pl.BlockSpec((D, K), lambda i: (0, 0)),
            pl.BlockSpec((K, D), lambda i: (0, 0)),
        ],
        out_specs=[
            pl.BlockSpec((BM, D), lambda i: (i, 0)),
            pl.BlockSpec((1, 1), lambda i: (0, 0)),
            pl.BlockSpec((1, 1), lambda i: (0, 0)),
        ],
        out_shape=[
            jax.ShapeDtypeStruct((N, D), jnp.float32),
            jax.ShapeDtypeStruct((1, 1), jnp.float32),
            jax.ShapeDtypeStruct((1, 1), jnp.float32),
        ],
        scratch_shapes=[
            pltpu.VMEM((D, K), jnp.bfloat16),
            pltpu.VMEM((1, K), jnp.float32),
            pltpu.VMEM((1, K), jnp.float32),
            pltpu.VMEM((1, 1), jnp.float32),
        ],
        compiler_params=pltpu.CompilerParams(
            dimension_semantics=("arbitrary",),
        ),
    )(z2d, cbt, cb16)

    return (zqst.reshape(z.shape), loss.reshape(()), perp.reshape(()))


# speedup vs baseline: 1.9072x; 1.9072x over previous
import functools

import jax
import jax.numpy as jnp
from jax import lax
from jax.experimental import pallas as pl
from jax.experimental.pallas import tpu as pltpu
from jax.experimental.pallas import tpu_sc as plsc

K = 8192
D = 256
N = 16384
BM = 256
NB = N // BM

NC = 2
NS = 16
NW = NC * NS
RPW = N // NW
CH = 128
NCH = RPW // CH
SL = K // NS


def _argmin_body(z_ref, cbt_ref, idx_ref, loss_ref, cbt16_s, c2_s, loss_s):
    i = pl.program_id(0)

    @pl.when(i == 0)
    def _init():
        cbt16_s[...] = cbt_ref[...].astype(jnp.bfloat16)
        c2_s[...] = jnp.sum(cbt_ref[...] * cbt_ref[...], axis=0, keepdims=True)
        loss_s[...] = jnp.zeros_like(loss_s)

    zb = z_ref[...]
    zb16 = zb.astype(jnp.bfloat16)
    p2 = 2.0 * jnp.dot(zb16, cbt16_s[...], preferred_element_type=jnp.float32)
    a = jnp.sum(zb * zb, axis=1, keepdims=True)
    d = (a - p2) + c2_s[...]

    m = jnp.min(d, axis=1, keepdims=True)
    lanes = jax.lax.broadcasted_iota(jnp.int32, (1, K), 1)
    idx = jnp.min(jnp.where(d == m, lanes, K), axis=1)
    idx_ref[...] = idx.reshape(1, 1, BM)
    loss_s[...] += jnp.sum(m)[None, None]

    @pl.when(i == NB - 1)
    def _fini():
        loss_ref[...] = 0.25 * loss_s[...] / (N * D)


def _sc_gather_hist(cb_hbm, idx_hbm, zq_hbm, cnt_hbm,
                    idx_v, buf_v, ones_v, slice_v, cnt_sh, sem):
    c = lax.axis_index("c")
    s = lax.axis_index("s")
    wid = s * NC + c
    base = wid * RPW

    pltpu.sync_copy(idx_hbm.at[pl.ds(base, RPW)], idx_v)
    for j in range(RPW // 16):
        ones_v[pl.ds(j * 16, 16)] = jnp.ones((16,), jnp.float32)
    for j in range(SL // 16):
        slice_v[pl.ds(j * 16, 16)] = jnp.zeros((16,), jnp.float32)

    pltpu.sync_copy(slice_v, cnt_sh.at[pl.ds(s * SL, SL)])
    plsc.subcore_barrier()

    pltpu.sync_copy(ones_v, cnt_sh.at[idx_v], add=True)

    for ch in range(NCH):
        idx_chunk = idx_v.at[pl.ds(ch * CH, CH)]
        pltpu.async_copy(cb_hbm.at[idx_chunk], buf_v, sem).wait()
        pltpu.sync_copy(buf_v, zq_hbm.at[pl.ds(base + ch * CH, CH)])

    plsc.subcore_barrier()
    pltpu.sync_copy(cnt_sh.at[pl.ds(s * SL, SL)], slice_v)
    pltpu.sync_copy(slice_v, cnt_hbm.at[c, pl.ds(s * SL, SL)])


def _perp_body(cnt_ref, perp_ref):
    p = (cnt_ref[0:1, :] + cnt_ref[1:2, :]) * (1.0 / N)
    perp_ref[...] = jnp.exp(-jnp.sum(p * jnp.log(p + 1e-05)))[None, None]


def kernel(z, codebook):
    z2d = z.reshape(N, D)
    cbt = codebook.T

    idx3, loss = pl.pallas_call(
        _argmin_body,
        grid=(NB,),
        in_specs=[
            pl.BlockSpec((BM, D), lambda i: (i, 0)),
            pl.BlockSpec((D, K), lambda i: (0, 0)),
        ],
        out_specs=[
            pl.BlockSpec((1, 1, BM), lambda i: (i, 0, 0)),
            pl.BlockSpec((1, 1), lambda i: (0, 0)),
        ],
        out_shape=[
            jax.ShapeDtypeStruct((NB, 1, BM), jnp.int32),
            jax.ShapeDtypeStruct((1, 1), jnp.float32),
        ],
        scratch_shapes=[
            pltpu.VMEM((D, K), jnp.bfloat16),
            pltpu.VMEM((1, K), jnp.float32),
            pltpu.VMEM((1, 1), jnp.float32),
        ],
        compiler_params=pltpu.CompilerParams(
            dimension_semantics=("arbitrary",),
        ),
    )(z2d, cbt)

    idx = idx3.reshape(N)

    sc = functools.partial(
        pl.kernel,
        mesh=plsc.VectorSubcoreMesh(core_axis_name="c", subcore_axis_name="s"),
        out_type=[
            jax.ShapeDtypeStruct((N, D), jnp.float32),
            jax.ShapeDtypeStruct((NC, K), jnp.float32),
        ],
        scratch_types=[
            pltpu.VMEM((RPW,), jnp.int32),
            pltpu.VMEM((CH, D), jnp.float32),
            pltpu.VMEM((RPW,), jnp.float32),
            pltpu.VMEM((SL,), jnp.float32),
            pltpu.VMEM_SHARED((K,), jnp.float32),
            pltpu.SemaphoreType.DMA,
        ],
    )
    zq2d, cnt2 = sc(_sc_gather_hist)(codebook, idx)

    perp = pl.pallas_call(
        _perp_body,
        grid=(1,),
        in_specs=[pl.BlockSpec((NC, K), lambda i: (0, 0))],
        out_specs=pl.BlockSpec((1, 1), lambda i: (0, 0)),
        out_shape=jax.ShapeDtypeStruct((1, 1), jnp.float32),
    )(cnt2)

    return (zq2d.reshape(z.shape), loss.reshape(()), perp.reshape(()))


# speedup vs baseline: 2.0856x; 1.0936x over previous
import functools

import jax
import jax.numpy as jnp
from jax import lax
from jax.experimental import pallas as pl
from jax.experimental.pallas import tpu as pltpu
from jax.experimental.pallas import tpu_sc as plsc

K = 8192
D = 256
N = 16384
BM = 256
NB = N // BM
IDXC = 128
IDXR = N // IDXC
RB = BM // IDXC

NC = 2
NS = 16
NW = NC * NS
RPW = N // NW
CH = 128
NCH = RPW // CH
SL = K // NS


def _argmin_body(z_ref, cbt_ref, idx_ref, loss_ref, cbt16_s, c2_s, loss_s):
    i = pl.program_id(0)

    @pl.when(i == 0)
    def _init():
        cbt16_s[...] = cbt_ref[...].astype(jnp.bfloat16)
        c2_s[...] = jnp.sum(cbt_ref[...] * cbt_ref[...], axis=0, keepdims=True)
        loss_s[...] = jnp.zeros_like(loss_s)

    zb = z_ref[...]
    z2b16 = (zb + zb).astype(jnp.bfloat16)
    p2 = jnp.dot(z2b16, cbt16_s[...], preferred_element_type=jnp.float32)
    a = jnp.sum(zb * zb, axis=1, keepdims=True)
    d = (a - p2) + c2_s[...]

    m = jnp.min(d, axis=1, keepdims=True)
    lanes = jax.lax.broadcasted_iota(jnp.int32, (1, K), 1).astype(jnp.float32)
    idxf = jnp.min(jnp.where(d == m, lanes, jnp.float32(K)), axis=1)
    idx_ref[pl.ds(i * RB, RB), :] = idxf.astype(jnp.int32).reshape(RB, IDXC)
    loss_s[...] += jnp.sum(m)[None, None]

    @pl.when(i == NB - 1)
    def _fini():
        loss_ref[...] = 0.25 * loss_s[...] / (N * D)


def _sc_gather_hist(cb_hbm, idx_hbm, zq_hbm, cnt_hbm,
                    idx_v, buf0, buf1, ones_v, slice_v, cnt_sh, sem0, sem1):
    c = lax.axis_index("c")
    s = lax.axis_index("s")
    wid = s * NC + c
    base = wid * RPW
    irow = wid * (RPW // IDXC)

    pltpu.sync_copy(idx_hbm.at[pl.ds(irow, RPW // IDXC)], idx_v)
    for j in range(SL // 16):
        slice_v[pl.ds(j * 16, 16)] = jnp.zeros((16,), jnp.float32)

    pltpu.sync_copy(slice_v, cnt_sh.at[pl.ds(s * SL, SL)])

    cps = [None] * NCH
    bufs = [buf0, buf1]
    sems = [sem0, sem1]
    cps[0] = pltpu.async_copy(cb_hbm.at[idx_v.at[0]], buf0, sem0)

    for j in range(CH // 16):
        ones_v[pl.ds(j * 16, 16)] = jnp.ones((16,), jnp.float32)
    plsc.subcore_barrier()

    for ch in range(NCH):
        pltpu.sync_copy(ones_v, cnt_sh.at[idx_v.at[ch]], add=True)

    for ch in range(NCH):
        if ch + 1 < NCH:
            cps[ch + 1] = pltpu.async_copy(
                cb_hbm.at[idx_v.at[ch + 1]], bufs[(ch + 1) % 2], sems[(ch + 1) % 2])
        cps[ch].wait()
        pltpu.sync_copy(bufs[ch % 2], zq_hbm.at[pl.ds(base + ch * CH, CH)])

    plsc.subcore_barrier()
    pltpu.sync_copy(cnt_sh.at[pl.ds(s * SL, SL)], slice_v)
    pltpu.sync_copy(slice_v, cnt_hbm.at[c, pl.ds(s * SL, SL)])


def _perp_body(cnt_ref, perp_ref):
    p = (cnt_ref[0:1, :] + cnt_ref[1:2, :]) * (1.0 / N)
    perp_ref[...] = jnp.exp(-jnp.sum(p * jnp.log(p + 1e-05)))[None, None]


def kernel(z, codebook):
    z2d = z.reshape(N, D)
    cbt = codebook.T

    idx2d, loss = pl.pallas_call(
        _argmin_body,
        grid=(NB,),
        in_specs=[
            pl.BlockSpec((BM, D), lambda i: (i, 0)),
            pl.BlockSpec((D, K), lambda i: (0, 0)),
        ],
        out_specs=[
            pl.BlockSpec((IDXR, IDXC), lambda i: (0, 0)),
            pl.BlockSpec((1, 1), lambda i: (0, 0)),
        ],
        out_shape=[
            jax.ShapeDtypeStruct((IDXR, IDXC), jnp.int32),
            jax.ShapeDtypeStruct((1, 1), jnp.float32),
        ],
        scratch_shapes=[
            pltpu.VMEM((D, K), jnp.bfloat16),
            pltpu.VMEM((1, K), jnp.float32),
            pltpu.VMEM((1, 1), jnp.float32),
        ],
        compiler_params=pltpu.CompilerParams(
            dimension_semantics=("arbitrary",),
        ),
    )(z2d, cbt)

    sc = functools.partial(
        pl.kernel,
        mesh=plsc.VectorSubcoreMesh(core_axis_name="c", subcore_axis_name="s"),
        out_type=[
            jax.ShapeDtypeStruct((N, D), jnp.float32),
            jax.ShapeDtypeStruct((NC, K), jnp.float32),
        ],
        scratch_types=[
            pltpu.VMEM((NCH, CH), jnp.int32),
            pltpu.VMEM((CH, D), jnp.float32),
            pltpu.VMEM((CH, D), jnp.float32),
            pltpu.VMEM((CH,), jnp.float32),
            pltpu.VMEM((SL,), jnp.float32),
            pltpu.VMEM_SHARED((K,), jnp.float32),
            pltpu.SemaphoreType.DMA,
            pltpu.SemaphoreType.DMA,
        ],
    )
    zq2d, cnt2 = sc(_sc_gather_hist)(codebook, idx2d)

    perp = pl.pallas_call(
        _perp_body,
        grid=(1,),
        in_specs=[pl.BlockSpec((NC, K), lambda i: (0, 0))],
        out_specs=pl.BlockSpec((1, 1), lambda i: (0, 0)),
        out_shape=jax.ShapeDtypeStruct((1, 1), jnp.float32),
    )(cnt2)

    return (zq2d.reshape(z.shape), loss.reshape(()), perp.reshape(()))


# speedup vs baseline: 2.2005x; 1.0551x over previous
import functools

import jax
import jax.numpy as jnp
from jax import lax
from jax.experimental import pallas as pl
from jax.experimental.pallas import tpu as pltpu
from jax.experimental.pallas import tpu_sc as plsc

K = 8192
D = 256
N = 16384
BM = 512
NB = N // BM
IDXC = 128
IDXR = N // IDXC
RB = BM // IDXC

NC = 2
NS = 16
NW = NC * NS
RPW = N // NW
CH = 128
NCH = RPW // CH
SL = K // NS


def _argmin_body(z_ref, cbt_ref, idx_ref, loss_ref, cbt16_s, c2_s, loss_s):
    i = pl.program_id(0)

    @pl.when(i == 0)
    def _init():
        cbt16_s[...] = cbt_ref[...].astype(jnp.bfloat16)
        c2_s[...] = jnp.sum(cbt_ref[...] * cbt_ref[...], axis=0, keepdims=True)
        loss_s[...] = jnp.zeros_like(loss_s)

    zb = z_ref[...]
    z2b16 = (zb + zb).astype(jnp.bfloat16)
    p2 = jnp.dot(z2b16, cbt16_s[...], preferred_element_type=jnp.float32)
    a = jnp.sum(zb * zb, axis=1, keepdims=True)
    d = (a - p2) + c2_s[...]

    m = jnp.min(d, axis=1, keepdims=True)
    lanes = jax.lax.broadcasted_iota(jnp.int32, (1, K), 1).astype(jnp.float32)
    idxf = jnp.min(jnp.where(d == m, lanes, jnp.float32(K)), axis=1)
    idx_ref[pl.ds(i * RB, RB), :] = idxf.astype(jnp.int32).reshape(RB, IDXC)
    loss_s[...] += jnp.sum(m)[None, None]

    @pl.when(i == NB - 1)
    def _fini():
        loss_ref[...] = 0.25 * loss_s[...] / (N * D)


def _sc_gather_hist(cb_hbm, idx_hbm, zq_hbm, cnt_hbm,
                    idx_v, buf0, buf1, ones_v, slice_v, cnt_sh, sem0, sem1):
    c = lax.axis_index("c")
    s = lax.axis_index("s")
    wid = s * NC + c
    base = wid * RPW
    irow = wid * (RPW // IDXC)

    pltpu.sync_copy(idx_hbm.at[pl.ds(irow, RPW // IDXC)], idx_v)
    for j in range(SL // 16):
        slice_v[pl.ds(j * 16, 16)] = jnp.zeros((16,), jnp.float32)

    pltpu.sync_copy(slice_v, cnt_sh.at[pl.ds(s * SL, SL)])

    cps = [None] * NCH
    bufs = [buf0, buf1]
    sems = [sem0, sem1]
    cps[0] = pltpu.async_copy(cb_hbm.at[idx_v.at[0]], buf0, sem0)

    for j in range(CH // 16):
        ones_v[pl.ds(j * 16, 16)] = jnp.ones((16,), jnp.float32)
    plsc.subcore_barrier()

    for ch in range(NCH):
        pltpu.sync_copy(ones_v, cnt_sh.at[idx_v.at[ch]], add=True)

    for ch in range(NCH):
        if ch + 1 < NCH:
            cps[ch + 1] = pltpu.async_copy(
                cb_hbm.at[idx_v.at[ch + 1]], bufs[(ch + 1) % 2], sems[(ch + 1) % 2])
        cps[ch].wait()
        pltpu.sync_copy(bufs[ch % 2], zq_hbm.at[pl.ds(base + ch * CH, CH)])

    plsc.subcore_barrier()
    pltpu.sync_copy(cnt_sh.at[pl.ds(s * SL, SL)], slice_v)
    pltpu.sync_copy(slice_v, cnt_hbm.at[c, pl.ds(s * SL, SL)])


def _perp_body(cnt_ref, perp_ref):
    p = (cnt_ref[0:1, :] + cnt_ref[1:2, :]) * (1.0 / N)
    perp_ref[...] = jnp.exp(-jnp.sum(p * jnp.log(p + 1e-05)))[None, None]


def kernel(z, codebook):
    z2d = z.reshape(N, D)
    cbt = codebook.T

    idx2d, loss = pl.pallas_call(
        _argmin_body,
        grid=(NB,),
        in_specs=[
            pl.BlockSpec((BM, D), lambda i: (i, 0)),
            pl.BlockSpec((D, K), lambda i: (0, 0)),
        ],
        out_specs=[
            pl.BlockSpec((IDXR, IDXC), lambda i: (0, 0)),
            pl.BlockSpec((1, 1), lambda i: (0, 0)),
        ],
        out_shape=[
            jax.ShapeDtypeStruct((IDXR, IDXC), jnp.int32),
            jax.ShapeDtypeStruct((1, 1), jnp.float32),
        ],
        scratch_shapes=[
            pltpu.VMEM((D, K), jnp.bfloat16),
            pltpu.VMEM((1, K), jnp.float32),
            pltpu.VMEM((1, 1), jnp.float32),
        ],
        compiler_params=pltpu.CompilerParams(
            dimension_semantics=("arbitrary",),
        ),
    )(z2d, cbt)

    sc = functools.partial(
        pl.kernel,
        mesh=plsc.VectorSubcoreMesh(core_axis_name="c", subcore_axis_name="s"),
        out_type=[
            jax.ShapeDtypeStruct((N, D), jnp.float32),
            jax.ShapeDtypeStruct((NC, K), jnp.float32),
        ],
        scratch_types=[
            pltpu.VMEM((NCH, CH), jnp.int32),
            pltpu.VMEM((CH, D), jnp.float32),
            pltpu.VMEM((CH, D), jnp.float32),
            pltpu.VMEM((CH,), jnp.float32),
            pltpu.VMEM((SL,), jnp.float32),
            pltpu.VMEM_SHARED((K,), jnp.float32),
            pltpu.SemaphoreType.DMA,
            pltpu.SemaphoreType.DMA,
        ],
    )
    zq2d, cnt2 = sc(_sc_gather_hist)(codebook, idx2d)

    perp = pl.pallas_call(
        _perp_body,
        grid=(1,),
        in_specs=[pl.BlockSpec((NC, K), lambda i: (0, 0))],
        out_specs=pl.BlockSpec((1, 1), lambda i: (0, 0)),
        out_shape=jax.ShapeDtypeStruct((1, 1), jnp.float32),
    )(cnt2)

    return (zq2d.reshape(z.shape), loss.reshape(()), perp.reshape(()))
